# baseline (speedup 1.0000x reference)
import functools
import jax
import jax.numpy as jnp
from jax import lax
from jax.experimental import pallas as pl
from jax.experimental.pallas import tpu as pltpu
from jax.experimental.pallas import tpu_sc as plsc

N = 10000
DEG = 32
D = 128
L = 16
NC = 2
NS = 16
NW = NC * NS
CHUNK = 8
NCHUNKS = N // CHUNK
BASE_CH = NCHUNKS // NW
EXTRA = NCHUNKS - BASE_CH * NW
ITERS = BASE_CH + 1


def _sc_body(ce_hbm, al_hbm, msg_hbm, out_hbm,
             msg0, msg1, al0, al1, ce0, ce1, ob0, ob1,
             isem0, isem1, osem0, osem1):
    wid = lax.axis_index("s") * NC + lax.axis_index("c")
    c0 = wid * BASE_CH + jnp.minimum(wid, EXTRA)
    nch = BASE_CH + jnp.where(wid < EXTRA, 1, 0)

    msgb = (msg0, msg1)
    alb = (al0, al1)
    ceb = (ce0, ce1)
    obb = (ob0, ob1)
    isems = (isem0, isem1)
    osems = (osem0, osem1)

    def row0(j):
        return (c0 + jnp.minimum(j, nch - 1)) * CHUNK

    def fire_in(j, b):
        n0 = row0(j)
        pltpu.make_async_copy(msg_hbm.at[pl.ds(n0, CHUNK)], msgb[b], isems[b]).start()
        pltpu.make_async_copy(al_hbm.at[pl.ds(n0, CHUNK)], alb[b], isems[b]).start()
        pltpu.make_async_copy(ce_hbm.at[pl.ds(n0, CHUNK)], ceb[b], isems[b]).start()

    def wait_in(j, b):
        n0 = row0(j)
        pltpu.make_async_copy(msg_hbm.at[pl.ds(n0, CHUNK)], msgb[b], isems[b]).wait()
        pltpu.make_async_copy(al_hbm.at[pl.ds(n0, CHUNK)], alb[b], isems[b]).wait()
        pltpu.make_async_copy(ce_hbm.at[pl.ds(n0, CHUNK)], ceb[b], isems[b]).wait()

    fire_in(0, 0)
    fire_in(1, 1)

    def group(it, _):
        for b in range(2):
            j = 2 * it + b
            wait_in(j, b)

            @pl.when(j >= 2)
            def _():
                n0p = row0(j - 2)
                pltpu.make_async_copy(obb[b], out_hbm.at[pl.ds(n0p, CHUNK)],
                                      osems[b]).wait()

            mv, av, cv, ov = msgb[b], alb[b], ceb[b], obb[b]

            def node(i, _):
                a = []
                for q in range(DEG // L):
                    aq = av[i, pl.ds(q * L, L)]
                    a.extend(aq[k] for k in range(L))
                for db in range(D // L):
                    acc = cv[i, pl.ds(db * L, L)]
                    for k in range(DEG):
                        acc = acc + a[k] * mv[i, pl.ds(k * D + db * L, L)]
                    ov[i, pl.ds(db * L, L)] = acc
                return 0

            lax.fori_loop(0, CHUNK, node, 0)

            n0 = row0(j)
            pltpu.make_async_copy(ov, out_hbm.at[pl.ds(n0, CHUNK)], osems[b]).start()

            @pl.when(j + 2 < ITERS)
            def _():
                fire_in(j + 2, b)
        return 0

    lax.fori_loop(0, ITERS // 2, group, 0)

    for b in range(2):
        n0 = row0(ITERS - 2 + b)
        pltpu.make_async_copy(obb[b], out_hbm.at[pl.ds(n0, CHUNK)], osems[b]).wait()


@jax.jit
def kernel(curr_emb, alpha, msg):
    ce = curr_emb[:, 0, :]
    al = alpha[:, :, 0]
    msgf = msg.reshape(N, DEG * D)

    mesh = plsc.VectorSubcoreMesh(core_axis_name="c", subcore_axis_name="s",
                                  num_cores=NC, num_subcores=NS)
    f = functools.partial(
        pl.kernel,
        out_type=jax.ShapeDtypeStruct((N, D), jnp.float32),
        mesh=mesh,
        scratch_types=[
            pltpu.VMEM((CHUNK, DEG * D), jnp.float32),
            pltpu.VMEM((CHUNK, DEG * D), jnp.float32),
            pltpu.VMEM((CHUNK, DEG), jnp.float32),
            pltpu.VMEM((CHUNK, DEG), jnp.float32),
            pltpu.VMEM((CHUNK, D), jnp.float32),
            pltpu.VMEM((CHUNK, D), jnp.float32),
            pltpu.VMEM((CHUNK, D), jnp.float32),
            pltpu.VMEM((CHUNK, D), jnp.float32),
            pltpu.SemaphoreType.DMA,
            pltpu.SemaphoreType.DMA,
            pltpu.SemaphoreType.DMA,
            pltpu.SemaphoreType.DMA,
        ],
    )(_sc_body)
    return f(ce, al, msgf)


# speedup vs baseline: 1.9037x; 1.9037x over previous
import functools
import jax
import jax.numpy as jnp
from jax import lax
from jax.experimental import pallas as pl
from jax.experimental.pallas import tpu as pltpu
from jax.experimental.pallas import tpu_sc as plsc

N = 10000
DEG = 32
D = 128
L = 16
NC = 2
NS = 16
NW = NC * NS
CHUNK = 8
NCHUNKS = N // CHUNK
BASE_CH = NCHUNKS // NW
EXTRA = NCHUNKS - BASE_CH * NW
ITERS = BASE_CH + 1


def _sc_body(ce_hbm, al_hbm, msg_hbm, out_hbm,
             msg0, msg1, al0, al1, ce0, ce1, ob0, ob1,
             isem0, isem1, osem0, osem1):
    wid = lax.axis_index("s") * NC + lax.axis_index("c")
    c0 = wid * BASE_CH + jnp.minimum(wid, EXTRA)
    nch = BASE_CH + jnp.where(wid < EXTRA, 1, 0)

    msgb = (msg0, msg1)
    alb = (al0, al1)
    ceb = (ce0, ce1)
    obb = (ob0, ob1)
    isems = (isem0, isem1)
    osems = (osem0, osem1)

    def row0(j):
        return (c0 + jnp.minimum(j, nch - 1)) * CHUNK

    def fire_in(j, b):
        n0 = row0(j)
        pltpu.make_async_copy(msg_hbm.at[pl.ds(n0, CHUNK)], msgb[b], isems[b]).start()
        pltpu.make_async_copy(al_hbm.at[pl.ds(n0, CHUNK)], alb[b], isems[b]).start()
        pltpu.make_async_copy(ce_hbm.at[pl.ds(n0, CHUNK), pl.ds(0, 1)], ceb[b], isems[b]).start()

    def wait_in(j, b):
        n0 = row0(j)
        pltpu.make_async_copy(msg_hbm.at[pl.ds(n0, CHUNK)], msgb[b], isems[b]).wait()
        pltpu.make_async_copy(al_hbm.at[pl.ds(n0, CHUNK)], alb[b], isems[b]).wait()
        pltpu.make_async_copy(ce_hbm.at[pl.ds(n0, CHUNK), pl.ds(0, 1)], ceb[b], isems[b]).wait()

    fire_in(0, 0)
    fire_in(1, 1)

    def group(it, _):
        for b in range(2):
            j = 2 * it + b
            wait_in(j, b)

            @pl.when(j >= 2)
            def _():
                n0p = row0(j - 2)
                pltpu.make_async_copy(obb[b], out_hbm.at[pl.ds(n0p, CHUNK)],
                                      osems[b]).wait()

            mv, av, cv, ov = msgb[b], alb[b], ceb[b], obb[b]

            def node(i, _):
                a = []
                for q in range(DEG // L):
                    aq = av[i, pl.ds(q * L, L)]
                    a.extend(aq[k] for k in range(L))
                for db in range(D // L):
                    acc = cv[i, 0, pl.ds(db * L, L)]
                    for k in range(DEG):
                        acc = acc + a[k] * mv[i, k, pl.ds(db * L, L)]
                    ov[i, pl.ds(db * L, L)] = acc
                return 0

            lax.fori_loop(0, CHUNK, node, 0)

            n0 = row0(j)
            pltpu.make_async_copy(ov, out_hbm.at[pl.ds(n0, CHUNK)], osems[b]).start()

            @pl.when(j + 2 < ITERS)
            def _():
                fire_in(j + 2, b)
        return 0

    lax.fori_loop(0, ITERS // 2, group, 0)

    for b in range(2):
        n0 = row0(ITERS - 2 + b)
        pltpu.make_async_copy(obb[b], out_hbm.at[pl.ds(n0, CHUNK)], osems[b]).wait()


@jax.jit
def kernel(curr_emb, alpha, msg):
    mesh = plsc.VectorSubcoreMesh(core_axis_name="c", subcore_axis_name="s",
                                  num_cores=NC, num_subcores=NS)
    f = functools.partial(
        pl.kernel,
        out_type=jax.ShapeDtypeStruct((N, D), jnp.float32),
        mesh=mesh,
        scratch_types=[
            pltpu.VMEM((CHUNK, DEG, D), jnp.float32),
            pltpu.VMEM((CHUNK, DEG, D), jnp.float32),
            pltpu.VMEM((CHUNK, DEG), jnp.float32),
            pltpu.VMEM((CHUNK, DEG), jnp.float32),
            pltpu.VMEM((CHUNK, 1, D), jnp.float32),
            pltpu.VMEM((CHUNK, 1, D), jnp.float32),
            pltpu.VMEM((CHUNK, D), jnp.float32),
            pltpu.VMEM((CHUNK, D), jnp.float32),
            pltpu.SemaphoreType.DMA,
            pltpu.SemaphoreType.DMA,
            pltpu.SemaphoreType.DMA,
            pltpu.SemaphoreType.DMA,
        ],
    )(_sc_body)
    return f(curr_emb, alpha[:, :, 0], msg)


# speedup vs baseline: 3.5154x; 1.8466x over previous
import functools
import jax
import jax.numpy as jnp
from jax import lax
from jax.experimental import pallas as pl
from jax.experimental.pallas import tpu as pltpu
from jax.experimental.pallas import tpu_sc as plsc

N = 10000
DEG = 32
D = 128
L = 16
NC = 2
NS = 16
NW = NC * NS
CHUNK = 8

NSC = 0
BN = 200
SUB = BN // 8


def _make_sc_body(nsc):
    nchunks = nsc // CHUNK
    base_ch = nchunks // NW
    extra = nchunks - base_ch * NW
    iters = base_ch + (1 if extra else 0)
    if iters % 2:
        iters += 1

    def body(ce_hbm, al_hbm, msg_hbm, out_hbm,
             msg0, msg1, al0, al1, ce0, ce1, ob0, ob1,
             isem0, isem1, osem0, osem1):
        wid = lax.axis_index("s") * NC + lax.axis_index("c")
        c0 = wid * base_ch + jnp.minimum(wid, extra)
        nch = base_ch + jnp.where(wid < extra, 1, 0)

        msgb = (msg0, msg1)
        alb = (al0, al1)
        ceb = (ce0, ce1)
        obb = (ob0, ob1)
        isems = (isem0, isem1)
        osems = (osem0, osem1)

        def row0(j):
            return (c0 + jnp.clip(j, 0, nch - 1)) * CHUNK

        def fire_in(j, b):
            n0 = row0(j)
            pltpu.make_async_copy(msg_hbm.at[pl.ds(n0, CHUNK)], msgb[b], isems[b]).start()
            pltpu.make_async_copy(al_hbm.at[pl.ds(n0, CHUNK)], alb[b], isems[b]).start()
            pltpu.make_async_copy(ce_hbm.at[pl.ds(n0, CHUNK), pl.ds(0, 1)], ceb[b], isems[b]).start()

        def wait_in(j, b):
            n0 = row0(j)
            pltpu.make_async_copy(msg_hbm.at[pl.ds(n0, CHUNK)], msgb[b], isems[b]).wait()
            pltpu.make_async_copy(al_hbm.at[pl.ds(n0, CHUNK)], alb[b], isems[b]).wait()
            pltpu.make_async_copy(ce_hbm.at[pl.ds(n0, CHUNK), pl.ds(0, 1)], ceb[b], isems[b]).wait()

        fire_in(0, 0)
        fire_in(1, 1)

        def group(it, _):
            for b in range(2):
                j = 2 * it + b
                wait_in(j, b)

                @pl.when(j >= 2)
                def _():
                    n0p = row0(j - 2)
                    pltpu.make_async_copy(obb[b], out_hbm.at[pl.ds(n0p, CHUNK)],
                                          osems[b]).wait()

                mv, av, cv, ov = msgb[b], alb[b], ceb[b], obb[b]

                def node(i, _):
                    a = []
                    for q in range(DEG // L):
                        aq = av[i, pl.ds(q * L, L)]
                        a.extend(aq[k] for k in range(L))
                    for db in range(D // L):
                        acc = cv[i, 0, pl.ds(db * L, L)]
                        for k in range(DEG):
                            acc = acc + a[k] * mv[i, k, pl.ds(db * L, L)]
                        ov[i, pl.ds(db * L, L)] = acc
                    return 0

                lax.fori_loop(0, CHUNK, node, 0)

                n0 = row0(j)
                pltpu.make_async_copy(ov, out_hbm.at[pl.ds(n0, CHUNK)], osems[b]).start()

                @pl.when(j + 2 < iters)
                def _():
                    fire_in(j + 2, b)
            return 0

        lax.fori_loop(0, iters // 2, group, 0)

        for b in range(2):
            n0 = row0(iters - 2 + b)
            pltpu.make_async_copy(obb[b], out_hbm.at[pl.ds(n0, CHUNK)], osems[b]).wait()

    return body


def _sc_part(curr_emb, alpha2, msg, nsc):
    mesh = plsc.VectorSubcoreMesh(core_axis_name="c", subcore_axis_name="s",
                                  num_cores=NC, num_subcores=NS)
    f = functools.partial(
        pl.kernel,
        out_type=jax.ShapeDtypeStruct((nsc, D), jnp.float32),
        mesh=mesh,
        scratch_types=[
            pltpu.VMEM((CHUNK, DEG, D), jnp.float32),
            pltpu.VMEM((CHUNK, DEG, D), jnp.float32),
            pltpu.VMEM((CHUNK, DEG), jnp.float32),
            pltpu.VMEM((CHUNK, DEG), jnp.float32),
            pltpu.VMEM((CHUNK, 1, D), jnp.float32),
            pltpu.VMEM((CHUNK, 1, D), jnp.float32),
            pltpu.VMEM((CHUNK, D), jnp.float32),
            pltpu.VMEM((CHUNK, D), jnp.float32),
            pltpu.SemaphoreType.DMA,
            pltpu.SemaphoreType.DMA,
            pltpu.SemaphoreType.DMA,
            pltpu.SemaphoreType.DMA,
        ],
    )(_make_sc_body(nsc))
    return f(curr_emb, alpha2, msg)


def _tc_body(al_ref, msg_ref, ce_hbm, out_ref, ceb, sem):
    i = pl.program_id(0)
    nblocks = pl.num_programs(0)

    def ce_copy(step, slot):
        n0 = NSC + step * BN
        return pltpu.make_async_copy(
            ce_hbm.at[pl.ds(n0, BN), pl.ds(0, 1)], ceb.at[slot], sem)

    @pl.when(i == 0)
    def _():
        ce_copy(0, 0).start()

    @pl.when(i + 1 < nblocks)
    def _():
        ce_copy(i + 1, (i + 1) % 2).start()

    ce_copy(i, i % 2).wait()

    bi = lax.broadcasted_iota(jnp.int32, (8, 8 * DEG), 0)
    ji = lax.broadcasted_iota(jnp.int32, (8, 8 * DEG), 1) // DEG
    for s in range(SUB):
        a8 = al_ref[pl.ds(8 * s, 8), :]
        tiled = jnp.concatenate([a8] * 8, axis=1)
        lhs = jnp.where(bi == ji, tiled, 0.0)
        r = jnp.dot(lhs.astype(jnp.bfloat16),
                    msg_ref[s].astype(jnp.bfloat16),
                    preferred_element_type=jnp.float32)
        out_ref[pl.ds(8 * s, 8), :] = r + ceb[i % 2, pl.ds(8 * s, 8), 0, :]


def _tc_part(curr_emb, alpha2, msg, nsc):
    ntc = N - nsc
    msg3 = msg.reshape(N // 8, 8 * DEG, D)
    grid = (ntc // BN,)
    off = nsc // BN
    return pl.pallas_call(
        _tc_body,
        grid=grid,
        in_specs=[
            pl.BlockSpec((BN, DEG), lambda i: (i + off, 0)),
            pl.BlockSpec((SUB, 8 * DEG, D), lambda i: (i + off, 0, 0)),
            pl.BlockSpec(memory_space=pltpu.HBM),
        ],
        out_specs=pl.BlockSpec((BN, D), lambda i: (i, 0)),
        out_shape=jax.ShapeDtypeStruct((ntc, D), jnp.float32),
        scratch_shapes=[
            pltpu.VMEM((2, BN, 1, D), jnp.float32),
            pltpu.SemaphoreType.DMA,
        ],
    )(alpha2, msg3, curr_emb)


@jax.jit
def kernel(curr_emb, alpha, msg):
    alpha2 = alpha[:, :, 0]
    parts = []
    if NSC > 0:
        parts.append(_sc_part(curr_emb, alpha2, msg, NSC))
    if NSC < N:
        parts.append(_tc_part(curr_emb, alpha2, msg, NSC))
    return parts[0] if len(parts) == 1 else jnp.concatenate(parts, axis=0)
